# baseline (speedup 1.0000x reference)
import jax
import jax.numpy as jnp
from jax import lax
from jax.experimental import pallas as pl
from jax.experimental.pallas import tpu as pltpu
from jax.experimental.pallas import tpu_sc as plsc

_NC = 2
_NS = 16
_NW = _NC * _NS
_G = 16
_NBUF = 6
_PITCH = 1001
_UNROLL = 16

_LOG1P_C = (
    9.0837865e-08, 0.9999915, -0.49980116, 0.331334, -0.23919071,
    0.1647835, -0.09231377, 0.034418594, -0.0060748775,
)


def _log1p_poly(u):
    acc = jnp.full_like(u, _LOG1P_C[-1])
    for c in _LOG1P_C[-2::-1]:
        acc = acc * u + c
    return acc


def _sc_body(z_hbm, cond_hbm, out_hbm, buf, cond_v, in_sems, out_sems):
    N, K = z_hbm.shape
    rows_per_w = N // _NW
    nch = rows_per_w // _G
    nfull = K // 16
    toff = K - 16
    tail = K - nfull * 16

    wid = lax.axis_index("s") * _NC + lax.axis_index("c")
    base = wid * rows_per_w

    pltpu.sync_copy(cond_hbm.at[pl.ds(base, rows_per_w)], cond_v)

    def in_copy(g, b):
        return pltpu.make_async_copy(
            z_hbm.at[pl.ds(base + g * _G, _G), :],
            buf.at[b, :, pl.ds(0, K)],
            in_sems.at[b],
        )

    def out_copy(g, b):
        return pltpu.make_async_copy(
            buf.at[b, :, pl.ds(0, K)],
            out_hbm.at[pl.ds(base + g * _G, _G), :],
            out_sems.at[b],
        )

    lane = lax.iota(jnp.int32, 16)
    neg_inf16 = jnp.full((16,), -jnp.inf, jnp.float32)
    one16 = jnp.full((16,), 1, jnp.int32)
    tail_sel = lane >= (16 - tail)

    for g in range(_NBUF):
        in_copy(g, g).start()

    for g in range(nch):
        b = g % _NBUF
        bufb = buf.at[b]
        in_copy(g, b).wait()

        c_vec = cond_v[pl.ds(g * _G, _G)]
        valid = c_vec < K
        cc = jnp.minimum(c_vec, K - 1)

        t_vec = plsc.load_gather(bufb, [lane, cc])
        plsc.store_scatter(bufb, [lane, cc], neg_inf16, mask=valid)

        def colmax(j, carry):
            acc, cols = carry
            for _ in range(_UNROLL):
                acc = jnp.maximum(acc, plsc.load_gather(bufb, [lane, cols]))
                cols = cols + one16
            return acc, cols

        acc, cols = lax.fori_loop(
            0, K // _UNROLL, colmax,
            (neg_inf16, jnp.full((16,), 0, jnp.int32)),
        )
        for _ in range(K - (K // _UNROLL) * _UNROLL):
            acc = jnp.maximum(acc, plsc.load_gather(bufb, [lane, cols]))
            cols = cols + one16

        m2 = jnp.maximum(acc, jnp.float32(0.0))
        hi = jnp.maximum(t_vec, m2)
        lo = jnp.minimum(t_vec, m2)
        v = hi + _log1p_poly(jnp.exp(lo - hi))
        plsc.store_scatter(bufb, [lane, cc], v, mask=valid)

        anyk = jnp.any(jnp.logical_not(valid))

        @pl.when(anyk)
        def _(bufb=bufb, g=g):
            def row_body(s, carry):
                c_one = plsc.load_gather(
                    cond_v, [jnp.full((16,), g * _G + s, jnp.int32)]
                )
                c_s = jnp.max(c_one)

                @pl.when(c_s == K)
                def _():
                    def sp(j, carry2):
                        x = bufb[s, pl.ds(j * 16, 16)]
                        y = jnp.minimum(x, 0.0) - _log1p_poly(
                            jnp.exp(-jnp.abs(x))
                        )
                        bufb[s, pl.ds(j * 16, 16)] = y
                        return carry2

                    lax.fori_loop(0, nfull, sp, 0)
                    x = bufb[s, pl.ds(toff, 16)]
                    y = jnp.minimum(x, 0.0) - _log1p_poly(jnp.exp(-jnp.abs(x)))
                    bufb[s, pl.ds(toff, 16)] = jnp.where(tail_sel, y, x)

                return carry

            lax.fori_loop(0, _G, row_body, 0)

        out_copy(g, b).start()

        p = g + (_NBUF // 2)
        if _NBUF <= p < nch:
            pb = p % _NBUF
            out_copy(p - _NBUF, pb).wait()
            in_copy(p, pb).start()

    for g in range(max(nch - _NBUF, 0), nch):
        out_copy(g, g % _NBUF).wait()


def kernel(z, cond):
    N, K = z.shape
    mesh = plsc.VectorSubcoreMesh(
        core_axis_name="c", subcore_axis_name="s", num_cores=_NC,
        num_subcores=_NS,
    )
    f = pl.kernel(
        _sc_body,
        out_type=jax.ShapeDtypeStruct((N, K), z.dtype),
        mesh=mesh,
        scratch_types=[
            pltpu.VMEM((_NBUF, _G, _PITCH), jnp.float32),
            pltpu.VMEM((N // _NW,), jnp.int32),
            pltpu.SemaphoreType.DMA((_NBUF,)),
            pltpu.SemaphoreType.DMA((_NBUF,)),
        ],
        compiler_params=pltpu.CompilerParams(
            needs_layout_passes=False, use_tc_tiling_on_sc=False
        ),
    )
    return f(z, cond)


# speedup vs baseline: 1.0413x; 1.0413x over previous
import jax
import jax.numpy as jnp
from jax import lax
from jax.experimental import pallas as pl
from jax.experimental.pallas import tpu as pltpu
from jax.experimental.pallas import tpu_sc as plsc

_NC = 2
_NS = 16
_NW = _NC * _NS
_G = 16
_NBUF = 6
_PITCH = 1001
_UNROLL = 16

_LOG1P_C = (
    9.0837865e-08, 0.9999915, -0.49980116, 0.331334, -0.23919071,
    0.1647835, -0.09231377, 0.034418594, -0.0060748775,
)


def _log1p_poly(u):
    acc = jnp.full_like(u, _LOG1P_C[-1])
    for c in _LOG1P_C[-2::-1]:
        acc = acc * u + c
    return acc


def _sc_body(z_hbm, cond_hbm, out_hbm, buf, cond_v, in_sems, out_sems):
    N, K = z_hbm.shape
    rows_per_w = N // _NW
    nch = rows_per_w // _G
    nfull = K // 16
    toff = K - 16
    tail = K - nfull * 16

    wid = lax.axis_index("s") * _NC + lax.axis_index("c")
    base = wid * rows_per_w

    pltpu.sync_copy(cond_hbm.at[pl.ds(base, rows_per_w)], cond_v)

    def in_copy(g, b):
        return pltpu.make_async_copy(
            z_hbm.at[pl.ds(base + g * _G, _G), :],
            buf.at[b],
            in_sems.at[b],
        )

    def out_copy(g, b):
        return pltpu.make_async_copy(
            buf.at[b],
            out_hbm.at[pl.ds(base + g * _G, _G), :],
            out_sems.at[b],
        )

    lane = lax.iota(jnp.int32, 16)
    neg_inf16 = jnp.full((16,), -jnp.inf, jnp.float32)
    one16 = jnp.full((16,), 1, jnp.int32)
    tail_sel = lane >= (16 - tail)

    for g in range(_NBUF):
        in_copy(g, g).start()

    for g in range(nch):
        b = g % _NBUF
        bufb = buf.at[b]
        in_copy(g, b).wait()

        c_vec = cond_v[pl.ds(g * _G, _G)]
        valid = c_vec < K
        cc = jnp.minimum(c_vec, K - 1)

        t_vec = plsc.load_gather(bufb, [lane, cc])
        plsc.store_scatter(bufb, [lane, cc], neg_inf16, mask=valid)

        def colmax(j, carry):
            acc, cols = carry
            for _ in range(_UNROLL):
                acc = jnp.maximum(acc, plsc.load_gather(bufb, [lane, cols]))
                cols = cols + one16
            return acc, cols

        acc, cols = lax.fori_loop(
            0, K // _UNROLL, colmax,
            (neg_inf16, jnp.full((16,), 0, jnp.int32)),
        )
        for _ in range(K - (K // _UNROLL) * _UNROLL):
            acc = jnp.maximum(acc, plsc.load_gather(bufb, [lane, cols]))
            cols = cols + one16

        m2 = jnp.maximum(acc, jnp.float32(0.0))
        hi = jnp.maximum(t_vec, m2)
        lo = jnp.minimum(t_vec, m2)
        v = hi + _log1p_poly(jnp.exp(lo - hi))
        plsc.store_scatter(bufb, [lane, cc], v, mask=valid)

        anyk = jnp.any(jnp.logical_not(valid))

        @pl.when(anyk)
        def _(bufb=bufb, g=g):
            def row_body(s, carry):
                c_one = plsc.load_gather(
                    cond_v, [jnp.full((16,), g * _G + s, jnp.int32)]
                )
                c_s = jnp.max(c_one)

                @pl.when(c_s == K)
                def _():
                    def sp(j, carry2):
                        x = bufb[s, pl.ds(j * 16, 16)]
                        y = jnp.minimum(x, 0.0) - _log1p_poly(
                            jnp.exp(-jnp.abs(x))
                        )
                        bufb[s, pl.ds(j * 16, 16)] = y
                        return carry2

                    lax.fori_loop(0, nfull, sp, 0)
                    x = bufb[s, pl.ds(toff, 16)]
                    y = jnp.minimum(x, 0.0) - _log1p_poly(jnp.exp(-jnp.abs(x)))
                    bufb[s, pl.ds(toff, 16)] = jnp.where(tail_sel, y, x)

                return carry

            lax.fori_loop(0, _G, row_body, 0)

        out_copy(g, b).start()

        p = g + (_NBUF // 2)
        if _NBUF <= p < nch:
            pb = p % _NBUF
            out_copy(p - _NBUF, pb).wait()
            in_copy(p, pb).start()

    for g in range(max(nch - _NBUF, 0), nch):
        out_copy(g, g % _NBUF).wait()


def kernel(z, cond):
    N, K = z.shape
    mesh = plsc.VectorSubcoreMesh(
        core_axis_name="c", subcore_axis_name="s", num_cores=_NC,
        num_subcores=_NS,
    )
    f = pl.kernel(
        _sc_body,
        out_type=jax.ShapeDtypeStruct((N, K), z.dtype),
        mesh=mesh,
        scratch_types=[
            pltpu.VMEM((_NBUF, _G, K), jnp.float32),
            pltpu.VMEM((N // _NW,), jnp.int32),
            pltpu.SemaphoreType.DMA((_NBUF,)),
            pltpu.SemaphoreType.DMA((_NBUF,)),
        ],
        compiler_params=pltpu.CompilerParams(
            needs_layout_passes=False, use_tc_tiling_on_sc=True
        ),
    )
    return f(z, cond)


# speedup vs baseline: 1.9696x; 1.8916x over previous
import jax
import jax.numpy as jnp
from jax.experimental import pallas as pl
from jax.experimental.pallas import tpu as pltpu

_R = 1024


def _block_kernel(cond_ref, z_ref, out_ref):
    z = z_ref[...]
    c = cond_ref[...][:, 0]
    K = z.shape[1]
    cols = jax.lax.broadcasted_iota(jnp.int32, z.shape, 1)
    is_t = cols == c[:, None]
    neg_inf = jnp.float32(-jnp.inf)
    other_max = jnp.max(jnp.where(is_t, neg_inf, z), axis=1)
    m = jnp.maximum(other_max, jnp.float32(0.0))
    t = jnp.max(jnp.where(is_t, z, neg_inf), axis=1)
    v = jnp.logaddexp(t, m)
    out = jnp.where(is_t, v[:, None], z)
    krow = c == K
    any_k = jnp.any(krow)

    @pl.when(any_k)
    def _():
        out_ref[...] = jnp.where(krow[:, None], -jax.nn.softplus(-z), out)

    @pl.when(jnp.logical_not(any_k))
    def _():
        out_ref[...] = out


def kernel(z, cond):
    N, K = z.shape
    cond2 = cond.reshape(N, 1)
    grid = (N // _R,)
    return pl.pallas_call(
        _block_kernel,
        grid=grid,
        in_specs=[
            pl.BlockSpec((_R, 1), lambda i: (i, 0)),
            pl.BlockSpec((_R, K), lambda i: (i, 0)),
        ],
        out_specs=pl.BlockSpec((_R, K), lambda i: (i, 0)),
        out_shape=jax.ShapeDtypeStruct((N, K), z.dtype),
        compiler_params=pltpu.CompilerParams(
            dimension_semantics=("arbitrary",),
        ),
    )(cond2, z)


# speedup vs baseline: 2.0286x; 1.0299x over previous
import jax
import jax.numpy as jnp
from jax.experimental import pallas as pl
from jax.experimental.pallas import tpu as pltpu

_R = 512


def _block_kernel(cond_ref, z_ref, out_ref):
    z = z_ref[...]
    c = cond_ref[...][:, 0]
    K = z.shape[1]
    cols = jax.lax.broadcasted_iota(jnp.int32, z.shape, 1)
    is_t = cols == c[:, None]
    neg_inf = jnp.float32(-jnp.inf)
    other_max = jnp.max(jnp.where(is_t, neg_inf, z), axis=1)
    m = jnp.maximum(other_max, jnp.float32(0.0))
    t = jnp.max(jnp.where(is_t, z, neg_inf), axis=1)
    v = jnp.logaddexp(t, m)
    out = jnp.where(is_t, v[:, None], z)
    krow = c == K
    any_k = jnp.any(krow)

    @pl.when(any_k)
    def _():
        out_ref[...] = jnp.where(krow[:, None], -jax.nn.softplus(-z), out)

    @pl.when(jnp.logical_not(any_k))
    def _():
        out_ref[...] = out


def kernel(z, cond):
    N, K = z.shape
    cond2 = cond.reshape(N, 1)
    grid = (N // _R,)
    return pl.pallas_call(
        _block_kernel,
        grid=grid,
        in_specs=[
            pl.BlockSpec((_R, 1), lambda i: (i, 0)),
            pl.BlockSpec((_R, K), lambda i: (i, 0)),
        ],
        out_specs=pl.BlockSpec((_R, K), lambda i: (i, 0)),
        out_shape=jax.ShapeDtypeStruct((N, K), z.dtype),
        compiler_params=pltpu.CompilerParams(
            dimension_semantics=("arbitrary",),
        ),
    )(cond2, z)
